# baseline (speedup 1.0000x reference)
import functools

import jax
import jax.numpy as jnp
from jax import lax
from jax.experimental import pallas as pl
from jax.experimental.pallas import tpu as pltpu
from jax.experimental.pallas import tpu_sc as plsc

D = 32
CH = 100
RPB = 4
NBUF = 2

_info = plsc.get_sparse_core_info()
NC = _info.num_cores
NS = _info.num_subcores
NW = NC * NS


def _make_gather(batch: int, hist: int):
    blk_rows = RPB * hist
    cpb = blk_rows // CH
    assert cpb * CH == blk_rows
    assert batch % (NW * RPB) == 0
    rows_w = batch // NW
    nblk = rows_w // RPB
    assert nblk % NBUF == 0
    n_chunks = batch * hist // CH

    mesh = plsc.VectorSubcoreMesh(core_axis_name="c", subcore_axis_name="s")

    @functools.partial(
        pl.kernel,
        mesh=mesh,
        compiler_params=pltpu.CompilerParams(use_tc_tiling_on_sc=False),
        out_type=jax.ShapeDtypeStruct((batch, hist, D), jnp.float32),
        scratch_types=(
            [pltpu.VMEM((NBUF, cpb, CH), jnp.int32),
             pltpu.VMEM((NBUF, RPB, hist, D), jnp.float32)]
            + [pltpu.SemaphoreType.DMA] * NBUF
        ),
    )
    def gather_kernel(x_hbm, table_hbm, out_hbm, idx_v, rows_v, *gsems):
        cid = lax.axis_index("c")
        sid = lax.axis_index("s")
        wid = sid * NC + cid
        brow_base = wid * rows_w
        chunk_base = wid * (rows_w * hist // CH)

        def fire(blk, buf):
            pltpu.sync_copy(
                x_hbm.at[pl.ds(chunk_base + blk * cpb, cpb)], idx_v.at[buf]
            )
            for j in range(cpb):
                r, half = divmod(j * CH, hist)
                pltpu.async_copy(
                    table_hbm.at[idx_v.at[buf, j]],
                    rows_v.at[buf, r, pl.ds(half, CH)],
                    gsems[buf],
                )

        def drain(buf):
            pltpu.make_async_copy(
                out_hbm.at[pl.ds(0, RPB)], rows_v.at[buf], gsems[buf]
            ).wait()

        def write_out(blk, buf):
            pltpu.sync_copy(
                rows_v.at[buf],
                out_hbm.at[pl.ds(brow_base + blk * RPB, RPB)],
            )

        for b in range(NBUF):
            fire(b, b)

        def body(i, carry):
            for b in range(NBUF):
                blk = i * NBUF + b
                drain(b)
                write_out(blk, b)
                fire(blk + NBUF, b)
            return carry

        lax.fori_loop(0, nblk // NBUF - 1, body, 0)

        for b in range(NBUF):
            blk = (nblk - NBUF) + b
            drain(b)
            write_out(blk, b)

    return gather_kernel


def kernel(x, table):
    batch, hist = x.shape
    xf = x.reshape(batch * hist // CH, CH).astype(jnp.int32)
    return _make_gather(batch, hist)(xf, table)


# speedup vs baseline: 1.0083x; 1.0083x over previous
import functools

import jax
import jax.numpy as jnp
from jax import lax
from jax.experimental import pallas as pl
from jax.experimental.pallas import tpu as pltpu
from jax.experimental.pallas import tpu_sc as plsc

D = 32
CH = 128
CPB = 8
BLK = CH * CPB
NBUF = 2

_info = plsc.get_sparse_core_info()
NC = _info.num_cores
NS = _info.num_subcores
NW = NC * NS


def _make_gather(n_rows: int):
    assert n_rows % (NW * BLK) == 0
    b_per_w = n_rows // NW
    nblk = b_per_w // BLK
    assert nblk % NBUF == 0
    chunks_per_w = b_per_w // CH

    mesh = plsc.VectorSubcoreMesh(core_axis_name="c", subcore_axis_name="s")

    @functools.partial(
        pl.kernel,
        mesh=mesh,
        compiler_params=pltpu.CompilerParams(use_tc_tiling_on_sc=False),
        out_type=jax.ShapeDtypeStruct((n_rows, D), jnp.float32),
        scratch_types=(
            [pltpu.VMEM((NBUF, CPB, CH), jnp.int32),
             pltpu.VMEM((NBUF, BLK, D), jnp.float32)]
            + [pltpu.SemaphoreType.DMA] * NBUF
        ),
    )
    def gather_kernel(x_hbm, table_hbm, out_hbm, idx_v, rows_v, *gsems):
        cid = lax.axis_index("c")
        sid = lax.axis_index("s")
        wid = sid * NC + cid
        row_base = wid * b_per_w
        chunk_base = wid * chunks_per_w

        def fire(blk, buf):
            pltpu.sync_copy(
                x_hbm.at[pl.ds(chunk_base + blk * CPB, CPB)], idx_v.at[buf]
            )
            for j in range(CPB):
                pltpu.async_copy(
                    table_hbm.at[idx_v.at[buf, j]],
                    rows_v.at[buf, pl.ds(j * CH, CH)],
                    gsems[buf],
                )

        def drain(buf):
            pltpu.make_async_copy(
                out_hbm.at[pl.ds(0, BLK)], rows_v.at[buf], gsems[buf]
            ).wait()

        def write_out(blk, buf):
            pltpu.sync_copy(
                rows_v.at[buf], out_hbm.at[pl.ds(row_base + blk * BLK, BLK)]
            )

        for b in range(NBUF):
            fire(b, b)

        def body(i, carry):
            for b in range(NBUF):
                blk = i * NBUF + b
                drain(b)
                write_out(blk, b)
                fire(blk + NBUF, b)
            return carry

        lax.fori_loop(0, nblk // NBUF - 1, body, 0)

        for b in range(NBUF):
            blk = (nblk - NBUF) + b
            drain(b)
            write_out(blk, b)

    return gather_kernel


def kernel(x, table):
    batch, hist = x.shape
    n_rows = batch * hist
    xf = x.reshape(n_rows // CH, CH).astype(jnp.int32)
    out = _make_gather(n_rows)(xf, table)
    return out.reshape(batch, hist, D)


# speedup vs baseline: 1.1032x; 1.0940x over previous
import functools

import jax
import jax.numpy as jnp
from jax import lax
from jax.experimental import pallas as pl
from jax.experimental.pallas import tpu as pltpu
from jax.experimental.pallas import tpu_sc as plsc

D = 32
CH = 128
CPB = 8
BLK = CH * CPB
NBUF = 2

_info = plsc.get_sparse_core_info()
NC = _info.num_cores
NS = _info.num_subcores
NW = NC * NS


def _make_gather(n_rows: int):
    assert n_rows % (NW * BLK) == 0
    b_per_w = n_rows // NW
    nblk = b_per_w // BLK
    assert nblk % NBUF == 0
    chunks_per_w = b_per_w // CH

    mesh = plsc.VectorSubcoreMesh(core_axis_name="c", subcore_axis_name="s")

    @functools.partial(
        pl.kernel,
        mesh=mesh,
        compiler_params=pltpu.CompilerParams(use_tc_tiling_on_sc=False),
        out_type=jax.ShapeDtypeStruct((n_rows, D), jnp.float32),
        scratch_types=(
            [pltpu.VMEM((NBUF, CPB, CH), jnp.int32),
             pltpu.VMEM((NBUF, BLK, D), jnp.float32)]
            + [pltpu.SemaphoreType.DMA] * NBUF
        ),
    )
    def gather_kernel(x_hbm, table_hbm, out_hbm, idx_v, rows_v, *gsems):
        cid = lax.axis_index("c")
        sid = lax.axis_index("s")
        wid = sid * NC + cid
        row_base = wid * b_per_w
        chunk_base = wid * chunks_per_w

        def fire(blk, buf):
            pltpu.sync_copy(
                x_hbm.at[pl.ds(chunk_base + blk * CPB, CPB)], idx_v.at[buf]
            )
            for j in range(CPB):
                pltpu.async_copy(
                    table_hbm.at[idx_v.at[buf, j]],
                    rows_v.at[buf, pl.ds(j * CH, CH)],
                    gsems[buf],
                )

        def drain(buf):
            pltpu.make_async_copy(
                out_hbm.at[pl.ds(0, BLK)], rows_v.at[buf], gsems[buf]
            ).wait()

        def write_out(blk, buf):
            pltpu.sync_copy(
                rows_v.at[buf], out_hbm.at[pl.ds(row_base + blk * BLK, BLK)]
            )

        for b in range(NBUF):
            fire(b, b)

        def body(i, carry):
            for b in range(NBUF):
                blk = i * NBUF + b
                drain(b)
                write_out(blk, b)
                fire(blk + NBUF, b)
            return carry

        lax.fori_loop(0, nblk // NBUF - 1, body, 0)

        for b in range(NBUF):
            blk = (nblk - NBUF) + b
            drain(b)
            write_out(blk, b)

    return gather_kernel


def kernel(x, table):
    batch, hist = x.shape
    n_rows = batch * hist
    xf = x.T.reshape(n_rows // CH, CH).astype(jnp.int32)
    out = _make_gather(n_rows)(xf, table)
    return out.reshape(hist, batch, D).transpose(1, 0, 2)


# speedup vs baseline: 2.0695x; 1.8760x over previous
import functools

import jax
import jax.numpy as jnp
from jax import lax
from jax.experimental import pallas as pl
from jax.experimental.pallas import tpu as pltpu
from jax.experimental.pallas import tpu_sc as plsc

D = 32
CH = 128
CPB = 8
BLK = CH * CPB
NBUF = 2

_info = plsc.get_sparse_core_info()
NC = _info.num_cores
NS = _info.num_subcores
NW = NC * NS


def _make_gather(n_rows: int):
    assert n_rows % (NW * BLK) == 0
    b_per_w = n_rows // NW
    nblk = b_per_w // BLK
    assert nblk % NBUF == 0
    chunks_per_w = b_per_w // CH

    mesh = plsc.VectorSubcoreMesh(core_axis_name="c", subcore_axis_name="s")

    @functools.partial(
        pl.kernel,
        mesh=mesh,
        compiler_params=pltpu.CompilerParams(use_tc_tiling_on_sc=False),
        out_type=jax.ShapeDtypeStruct((n_rows, 128), jnp.float32),
        scratch_types=(
            [pltpu.VMEM((NBUF, CPB, CH), jnp.int32),
             pltpu.VMEM((NBUF, BLK, D), jnp.float32)]
            + [pltpu.SemaphoreType.DMA] * NBUF
        ),
    )
    def gather_kernel(x_hbm, table_hbm, out_hbm, idx_v, rows_v, *gsems):
        cid = lax.axis_index("c")
        sid = lax.axis_index("s")
        wid = sid * NC + cid
        row_base = wid * b_per_w
        chunk_base = wid * chunks_per_w

        def fire(blk, buf):
            pltpu.sync_copy(
                x_hbm.at[pl.ds(chunk_base + blk * CPB, CPB)], idx_v.at[buf]
            )
            for j in range(CPB):
                pltpu.async_copy(
                    table_hbm.at[idx_v.at[buf, j]],
                    rows_v.at[buf, pl.ds(j * CH, CH)],
                    gsems[buf],
                )

        def drain(buf):
            pltpu.make_async_copy(
                out_hbm.at[pl.ds(0, BLK), pl.ds(0, D)], rows_v.at[buf],
                gsems[buf],
            ).wait()

        def write_out(blk, buf):
            pltpu.sync_copy(
                rows_v.at[buf],
                out_hbm.at[pl.ds(row_base + blk * BLK, BLK), pl.ds(0, D)],
            )

        for b in range(NBUF):
            fire(b, b)

        def body(i, carry):
            for b in range(NBUF):
                blk = i * NBUF + b
                drain(b)
                write_out(blk, b)
                fire(blk + NBUF, b)
            return carry

        lax.fori_loop(0, nblk // NBUF - 1, body, 0)

        for b in range(NBUF):
            blk = (nblk - NBUF) + b
            drain(b)
            write_out(blk, b)

    return gather_kernel


def kernel(x, table):
    batch, hist = x.shape
    n_rows = batch * hist
    xf = x.T.reshape(n_rows // CH, CH).astype(jnp.int32)
    out = _make_gather(n_rows)(xf, table)
    return out[:, :D].reshape(hist, batch, D).transpose(1, 0, 2)


# speedup vs baseline: 2.0730x; 1.0017x over previous
import functools

import jax
import jax.numpy as jnp
from jax import lax
from jax.experimental import pallas as pl
from jax.experimental.pallas import tpu as pltpu
from jax.experimental.pallas import tpu_sc as plsc

D = 32
CH = 512
CPB = 2
BLK = CH * CPB
NBUF = 2

_info = plsc.get_sparse_core_info()
NC = _info.num_cores
NS = _info.num_subcores
NW = NC * NS


def _make_gather(n_rows: int):
    assert n_rows % (NW * BLK) == 0
    b_per_w = n_rows // NW
    nblk = b_per_w // BLK
    assert nblk % NBUF == 0
    chunks_per_w = b_per_w // CH

    mesh = plsc.VectorSubcoreMesh(core_axis_name="c", subcore_axis_name="s")

    @functools.partial(
        pl.kernel,
        mesh=mesh,
        compiler_params=pltpu.CompilerParams(use_tc_tiling_on_sc=False),
        out_type=jax.ShapeDtypeStruct((n_rows, 128), jnp.float32),
        scratch_types=(
            [pltpu.VMEM((NBUF, CPB, CH), jnp.int32),
             pltpu.VMEM((NBUF, BLK, D), jnp.float32)]
            + [pltpu.SemaphoreType.DMA] * NBUF
        ),
    )
    def gather_kernel(x_hbm, table_hbm, out_hbm, idx_v, rows_v, *gsems):
        cid = lax.axis_index("c")
        sid = lax.axis_index("s")
        wid = sid * NC + cid
        row_base = wid * b_per_w
        chunk_base = wid * chunks_per_w

        def fire(blk, buf):
            pltpu.sync_copy(
                x_hbm.at[pl.ds(chunk_base + blk * CPB, CPB)], idx_v.at[buf]
            )
            for j in range(CPB):
                pltpu.async_copy(
                    table_hbm.at[idx_v.at[buf, j]],
                    rows_v.at[buf, pl.ds(j * CH, CH)],
                    gsems[buf],
                )

        def drain(buf):
            pltpu.make_async_copy(
                out_hbm.at[pl.ds(0, BLK), pl.ds(0, D)], rows_v.at[buf],
                gsems[buf],
            ).wait()

        def write_out(blk, buf):
            pltpu.sync_copy(
                rows_v.at[buf],
                out_hbm.at[pl.ds(row_base + blk * BLK, BLK), pl.ds(0, D)],
            )

        for b in range(NBUF):
            fire(b, b)

        def body(i, carry):
            for b in range(NBUF):
                blk = i * NBUF + b
                drain(b)
                write_out(blk, b)
                fire(blk + NBUF, b)
            return carry

        lax.fori_loop(0, nblk // NBUF - 1, body, 0)

        for b in range(NBUF):
            blk = (nblk - NBUF) + b
            drain(b)
            write_out(blk, b)

    return gather_kernel


def kernel(x, table):
    batch, hist = x.shape
    n_rows = batch * hist
    xf = x.T.reshape(n_rows // CH, CH).astype(jnp.int32)
    out = _make_gather(n_rows)(xf, table)
    return out[:, :D].reshape(hist, batch, D).transpose(1, 0, 2)
